# Initial kernel scaffold; baseline (speedup 1.0000x reference)
import jax
import jax.numpy as jnp
from jax.experimental import pallas as pl


def kernel(x, edge_index, W0, b0, W1, b1):
    raise NotImplementedError("write your pallas kernel here")


# speedup vs baseline: 14.6786x; 14.6786x over previous
import functools

import jax
import jax.numpy as jnp
from jax import lax
from jax.experimental import pallas as pl
from jax.experimental.pallas import tpu as pltpu
from jax.experimental.pallas import tpu_sc as plsc

N = 10000
D = 128
E = 320000
NC = 2
NS = 16
NW = NC * NS
EW = E // NW
CH = 125
NCH = EW // CH
IW = 1000
NIW = N // IW

_mesh = plsc.VectorSubcoreMesh(core_axis_name="c", subcore_axis_name="s")


DW = 16


@functools.partial(
    pl.kernel,
    out_type=jax.ShapeDtypeStruct((NC, N, DW), jnp.float32),
    mesh=_mesh,
    scratch_types=[
        pltpu.VMEM((NCH, CH), jnp.int32),
        pltpu.VMEM((CH, DW), jnp.float32),
        pltpu.VMEM_SHARED((N, DW), jnp.float32),
    ],
    compiler_params=pltpu.CompilerParams(use_tc_tiling_on_sc=False),
)
def _deg_kernel(col_hbm, ones_hbm, zeros_hbm, out_hbm, colv, ones, acc):
    c = lax.axis_index("c")
    s = lax.axis_index("s")
    w = c * NS + s

    @pl.when(s < NIW)
    def _():
        pltpu.sync_copy(zeros_hbm.at[pl.ds(s * IW, IW)],
                        acc.at[pl.ds(s * IW, IW)])

    pltpu.sync_copy(col_hbm.at[w], colv)
    pltpu.sync_copy(ones_hbm, ones)
    plsc.subcore_barrier()

    def body(j, carry):
        pltpu.sync_copy(ones, acc.at[colv.at[j]], add=True)
        return carry

    lax.fori_loop(0, NCH, body, 0)
    plsc.subcore_barrier()

    @pl.when(s < NIW)
    def _():
        pltpu.sync_copy(acc.at[pl.ds(s * IW, IW)],
                        out_hbm.at[c, pl.ds(s * IW, IW)])


@functools.partial(
    pl.kernel,
    out_type=jax.ShapeDtypeStruct((NC, N, D), jnp.float32),
    mesh=_mesh,
    scratch_types=[
        pltpu.VMEM((NCH, CH), jnp.int32),
        pltpu.VMEM((NCH, CH), jnp.int32),
        pltpu.VMEM((CH, D), jnp.float32),
        pltpu.VMEM_SHARED((N, D), jnp.float32),
    ],
)
def _prop_kernel(h_hbm, row_hbm, col_hbm, zeros_hbm, out_hbm,
                 rowv, colv, gbuf, acc):
    c = lax.axis_index("c")
    s = lax.axis_index("s")
    w = c * NS + s

    @pl.when(s < NIW)
    def _():
        pltpu.sync_copy(zeros_hbm.at[pl.ds(s * IW, IW)],
                        acc.at[pl.ds(s * IW, IW)])

    pltpu.sync_copy(row_hbm.at[w], rowv)
    pltpu.sync_copy(col_hbm.at[w], colv)
    plsc.subcore_barrier()

    def body(j, carry):
        pltpu.sync_copy(h_hbm.at[rowv.at[j]], gbuf)
        pltpu.sync_copy(gbuf, acc.at[colv.at[j]], add=True)
        return carry

    lax.fori_loop(0, NCH, body, 0)
    plsc.subcore_barrier()

    @pl.when(s < NIW)
    def _():
        pltpu.sync_copy(acc.at[pl.ds(s * IW, IW)],
                        out_hbm.at[c, pl.ds(s * IW, IW)])


BR = 2000


def _dis_block(deg_ref):
    degs = deg_ref[:, 0:1] + deg_ref[:, 1:2]
    return jnp.where(degs > 0, lax.rsqrt(degs), 0.0)


def _tc_pre_body(x_ref, deg_ref, w_ref, o_ref):
    dis = _dis_block(deg_ref)
    o_ref[...] = jnp.dot(x_ref[...] * dis, w_ref[...],
                         preferred_element_type=jnp.float32)


def _tc_pre(x, deg_t, w):
    return pl.pallas_call(
        _tc_pre_body,
        grid=(N // BR,),
        in_specs=[
            pl.BlockSpec((BR, D), lambda i: (i, 0)),
            pl.BlockSpec((BR, NC), lambda i: (i, 0)),
            pl.BlockSpec((D, D), lambda i: (0, 0)),
        ],
        out_specs=pl.BlockSpec((BR, D), lambda i: (i, 0)),
        out_shape=jax.ShapeDtypeStruct((N, D), jnp.float32),
    )(x, deg_t, w)


def _l2n(g):
    nrm = jnp.sqrt(jnp.sum(g * g, axis=1, keepdims=True))
    return g / jnp.maximum(nrm, 1e-12)


def _tc_mid_body(s_ref, deg_ref, b_ref, w_ref, g_ref, n_ref, hp_ref):
    dis = _dis_block(deg_ref)
    g = dis * (s_ref[0] + s_ref[1]) + b_ref[...]
    n = _l2n(g)
    g_ref[...] = g
    n_ref[...] = n
    hp_ref[...] = jnp.dot(n * dis, w_ref[...],
                          preferred_element_type=jnp.float32)


def _tc_mid(s1, deg_t, b0, w1):
    return pl.pallas_call(
        _tc_mid_body,
        grid=(N // BR,),
        in_specs=[
            pl.BlockSpec((NC, BR, D), lambda i: (0, i, 0)),
            pl.BlockSpec((BR, NC), lambda i: (i, 0)),
            pl.BlockSpec((1, D), lambda i: (0, 0)),
            pl.BlockSpec((D, D), lambda i: (0, 0)),
        ],
        out_specs=[
            pl.BlockSpec((BR, D), lambda i: (i, 0)),
            pl.BlockSpec((BR, D), lambda i: (i, 0)),
            pl.BlockSpec((BR, D), lambda i: (i, 0)),
        ],
        out_shape=[
            jax.ShapeDtypeStruct((N, D), jnp.float32),
            jax.ShapeDtypeStruct((N, D), jnp.float32),
            jax.ShapeDtypeStruct((N, D), jnp.float32),
        ],
    )(s1, deg_t, b0, w1)


def _tc_post_body(s_ref, deg_ref, b_ref, g1_ref, n1_ref, x_ref,
                  rs_ref, fin_ref, cl_ref):
    dis = _dis_block(deg_ref)
    g2 = dis * (s_ref[0] + s_ref[1]) + b_ref[...]
    n2 = _l2n(g2)
    rs_ref[...] = x_ref[...] + n1_ref[...] + 0.5 * n2
    fin_ref[...] = 0.5 * (g1_ref[...] + g2)
    cl_ref[...] = g2


def _tc_post(s2, deg_t, b1, g1, n1, x):
    return pl.pallas_call(
        _tc_post_body,
        grid=(N // BR,),
        in_specs=[
            pl.BlockSpec((NC, BR, D), lambda i: (0, i, 0)),
            pl.BlockSpec((BR, NC), lambda i: (i, 0)),
            pl.BlockSpec((1, D), lambda i: (0, 0)),
            pl.BlockSpec((BR, D), lambda i: (i, 0)),
            pl.BlockSpec((BR, D), lambda i: (i, 0)),
            pl.BlockSpec((BR, D), lambda i: (i, 0)),
        ],
        out_specs=[
            pl.BlockSpec((BR, D), lambda i: (i, 0)),
            pl.BlockSpec((BR, D), lambda i: (i, 0)),
            pl.BlockSpec((BR, D), lambda i: (i, 0)),
        ],
        out_shape=[
            jax.ShapeDtypeStruct((N, D), jnp.float32),
            jax.ShapeDtypeStruct((N, D), jnp.float32),
            jax.ShapeDtypeStruct((N, D), jnp.float32),
        ],
    )(s2, deg_t, b1, g1, n1, x)


def kernel(x, edge_index, W0, b0, W1, b1):
    row = edge_index[0].astype(jnp.int32).reshape(NW, NCH, CH)
    col = edge_index[1].astype(jnp.int32).reshape(NW, NCH, CH)
    o_c = jnp.ones((CH, DW), jnp.float32)
    z_n = jnp.zeros((N, DW), jnp.float32)
    z_nd = jnp.zeros((N, D), jnp.float32)

    deg2 = _deg_kernel(col, o_c, z_n)
    deg_t = deg2[:, :, 0].T

    hp0 = _tc_pre(x, deg_t, W0)
    s1 = _prop_kernel(hp0, row, col, z_nd)
    g1, n1, hp1 = _tc_mid(s1, deg_t, b0.reshape(1, D), W1)
    s2 = _prop_kernel(hp1, row, col, z_nd)
    rs, fin, cl = _tc_post(s2, deg_t, b1.reshape(1, D), g1, n1, x)
    return (rs, fin, cl)


# speedup vs baseline: 17.1869x; 1.1709x over previous
import functools

import jax
import jax.numpy as jnp
from jax import lax
from jax.experimental import pallas as pl
from jax.experimental.pallas import tpu as pltpu
from jax.experimental.pallas import tpu_sc as plsc

N = 10000
D = 128
E = 320000
NC = 2
NS = 16
NW = NC * NS
EW = E // NW
CHD = 250
NCHD = EW // CHD
CH = 125
NCH = EW // CH
STG = 5
CPS = NCH // STG
IW = 1000
NIW = N // IW

_mesh = plsc.VectorSubcoreMesh(core_axis_name="c", subcore_axis_name="s")


DW = 16


@functools.partial(
    pl.kernel,
    out_type=jax.ShapeDtypeStruct((NC, N, DW), jnp.float32),
    mesh=_mesh,
    scratch_types=[
        pltpu.VMEM((NCHD, CHD), jnp.int32),
        pltpu.VMEM((CHD, DW), jnp.float32),
        pltpu.VMEM_SHARED((N, DW), jnp.float32),
    ],
    compiler_params=pltpu.CompilerParams(use_tc_tiling_on_sc=False),
)
def _deg_kernel(col_hbm, ones_hbm, zeros_hbm, out_hbm, colv, ones, acc):
    c = lax.axis_index("c")
    s = lax.axis_index("s")
    w = c * NS + s

    @pl.when(s < NIW)
    def _():
        pltpu.sync_copy(zeros_hbm.at[pl.ds(s * IW, IW)],
                        acc.at[pl.ds(s * IW, IW)])

    pltpu.sync_copy(col_hbm.at[w], colv)
    pltpu.sync_copy(ones_hbm, ones)
    plsc.subcore_barrier()

    def body(j, carry):
        pltpu.sync_copy(ones, acc.at[colv.at[j]], add=True)
        return carry

    lax.fori_loop(0, NCHD, body, 0)
    plsc.subcore_barrier()

    @pl.when(s < NIW)
    def _():
        pltpu.sync_copy(acc.at[pl.ds(s * IW, IW)],
                        out_hbm.at[c, pl.ds(s * IW, IW)])


@functools.partial(
    pl.kernel,
    out_type=jax.ShapeDtypeStruct((NC, N, D), jnp.float32),
    mesh=_mesh,
    scratch_types=[
        pltpu.VMEM((CPS, CH), jnp.int32),
        pltpu.VMEM((CPS, CH), jnp.int32),
        pltpu.VMEM((CH, D), jnp.float32),
        pltpu.VMEM((CH, D), jnp.float32),
        pltpu.VMEM_SHARED((N, D), jnp.float32),
        pltpu.SemaphoreType.DMA,
    ],
)
def _prop_kernel(h_hbm, row_hbm, col_hbm, zeros_hbm, out_hbm,
                 rowv, colv, gb0, gb1, acc, gsem):
    c = lax.axis_index("c")
    s = lax.axis_index("s")
    w = c * NS + s

    @pl.when(s < NIW)
    def _():
        pltpu.sync_copy(zeros_hbm.at[pl.ds(s * IW, IW)],
                        acc.at[pl.ds(s * IW, IW)])

    plsc.subcore_barrier()

    for st in range(STG):
        pltpu.sync_copy(row_hbm.at[w, pl.ds(st * CPS, CPS)], rowv)
        pltpu.sync_copy(col_hbm.at[w, pl.ds(st * CPS, CPS)], colv)
        pltpu.async_copy(h_hbm.at[rowv.at[0]], gb0, gsem)

        def body(jj, carry):
            j0 = 2 * jj
            j1 = j0 + 1
            pltpu.make_async_copy(h_hbm.at[rowv.at[j0]], gb0, gsem).wait()
            pltpu.async_copy(h_hbm.at[rowv.at[j1]], gb1, gsem)
            pltpu.sync_copy(gb0, acc.at[colv.at[j0]], add=True)
            pltpu.make_async_copy(h_hbm.at[rowv.at[j1]], gb1, gsem).wait()

            @pl.when(j1 + 1 < CPS)
            def _():
                pltpu.async_copy(h_hbm.at[rowv.at[j1 + 1]], gb0, gsem)

            pltpu.sync_copy(gb1, acc.at[colv.at[j1]], add=True)
            return carry

        lax.fori_loop(0, CPS // 2, body, 0)
    plsc.subcore_barrier()

    @pl.when(s < NIW)
    def _():
        pltpu.sync_copy(acc.at[pl.ds(s * IW, IW)],
                        out_hbm.at[c, pl.ds(s * IW, IW)])


BR = 2000


def _dis_block(deg_ref):
    degs = deg_ref[:, 0:1] + deg_ref[:, 1:2]
    return jnp.where(degs > 0, lax.rsqrt(degs), 0.0)


def _tc_pre_body(x_ref, deg_ref, w_ref, o_ref):
    dis = _dis_block(deg_ref)
    o_ref[...] = jnp.dot(x_ref[...] * dis, w_ref[...],
                         preferred_element_type=jnp.float32)


def _tc_pre(x, deg_t, w):
    return pl.pallas_call(
        _tc_pre_body,
        grid=(N // BR,),
        in_specs=[
            pl.BlockSpec((BR, D), lambda i: (i, 0)),
            pl.BlockSpec((BR, NC), lambda i: (i, 0)),
            pl.BlockSpec((D, D), lambda i: (0, 0)),
        ],
        out_specs=pl.BlockSpec((BR, D), lambda i: (i, 0)),
        out_shape=jax.ShapeDtypeStruct((N, D), jnp.float32),
    )(x, deg_t, w)


def _l2n(g):
    nrm = jnp.sqrt(jnp.sum(g * g, axis=1, keepdims=True))
    return g / jnp.maximum(nrm, 1e-12)


def _tc_mid_body(s_ref, deg_ref, b_ref, w_ref, g_ref, n_ref, hp_ref):
    dis = _dis_block(deg_ref)
    g = dis * (s_ref[0] + s_ref[1]) + b_ref[...]
    n = _l2n(g)
    g_ref[...] = g
    n_ref[...] = n
    hp_ref[...] = jnp.dot(n * dis, w_ref[...],
                          preferred_element_type=jnp.float32)


def _tc_mid(s1, deg_t, b0, w1):
    return pl.pallas_call(
        _tc_mid_body,
        grid=(N // BR,),
        in_specs=[
            pl.BlockSpec((NC, BR, D), lambda i: (0, i, 0)),
            pl.BlockSpec((BR, NC), lambda i: (i, 0)),
            pl.BlockSpec((1, D), lambda i: (0, 0)),
            pl.BlockSpec((D, D), lambda i: (0, 0)),
        ],
        out_specs=[
            pl.BlockSpec((BR, D), lambda i: (i, 0)),
            pl.BlockSpec((BR, D), lambda i: (i, 0)),
            pl.BlockSpec((BR, D), lambda i: (i, 0)),
        ],
        out_shape=[
            jax.ShapeDtypeStruct((N, D), jnp.float32),
            jax.ShapeDtypeStruct((N, D), jnp.float32),
            jax.ShapeDtypeStruct((N, D), jnp.float32),
        ],
    )(s1, deg_t, b0, w1)


def _tc_post_body(s_ref, deg_ref, b_ref, g1_ref, n1_ref, x_ref,
                  rs_ref, fin_ref, cl_ref):
    dis = _dis_block(deg_ref)
    g2 = dis * (s_ref[0] + s_ref[1]) + b_ref[...]
    n2 = _l2n(g2)
    rs_ref[...] = x_ref[...] + n1_ref[...] + 0.5 * n2
    fin_ref[...] = 0.5 * (g1_ref[...] + g2)
    cl_ref[...] = g2


def _tc_post(s2, deg_t, b1, g1, n1, x):
    return pl.pallas_call(
        _tc_post_body,
        grid=(N // BR,),
        in_specs=[
            pl.BlockSpec((NC, BR, D), lambda i: (0, i, 0)),
            pl.BlockSpec((BR, NC), lambda i: (i, 0)),
            pl.BlockSpec((1, D), lambda i: (0, 0)),
            pl.BlockSpec((BR, D), lambda i: (i, 0)),
            pl.BlockSpec((BR, D), lambda i: (i, 0)),
            pl.BlockSpec((BR, D), lambda i: (i, 0)),
        ],
        out_specs=[
            pl.BlockSpec((BR, D), lambda i: (i, 0)),
            pl.BlockSpec((BR, D), lambda i: (i, 0)),
            pl.BlockSpec((BR, D), lambda i: (i, 0)),
        ],
        out_shape=[
            jax.ShapeDtypeStruct((N, D), jnp.float32),
            jax.ShapeDtypeStruct((N, D), jnp.float32),
            jax.ShapeDtypeStruct((N, D), jnp.float32),
        ],
    )(s2, deg_t, b1, g1, n1, x)


def kernel(x, edge_index, W0, b0, W1, b1):
    row = edge_index[0].astype(jnp.int32).reshape(NW, NCH, CH)
    col = edge_index[1].astype(jnp.int32).reshape(NW, NCH, CH)
    col_d = edge_index[1].astype(jnp.int32).reshape(NW, NCHD, CHD)
    o_c = jnp.ones((CHD, DW), jnp.float32)
    z_n = jnp.zeros((N, DW), jnp.float32)
    z_nd = jnp.zeros((N, D), jnp.float32)

    deg2 = _deg_kernel(col_d, o_c, z_n)
    deg_t = deg2[:, :, 0].T

    hp0 = _tc_pre(x, deg_t, W0)
    s1 = _prop_kernel(hp0, row, col, z_nd)
    g1, n1, hp1 = _tc_mid(s1, deg_t, b0.reshape(1, D), W1)
    s2 = _prop_kernel(hp1, row, col, z_nd)
    rs, fin, cl = _tc_post(s2, deg_t, b1.reshape(1, D), g1, n1, x)
    return (rs, fin, cl)
